# baseline (speedup 1.0000x reference)
import functools

import jax
import jax.numpy as jnp
from jax import lax
from jax.experimental import pallas as pl
from jax.experimental.pallas import tpu as pltpu
from jax.experimental.pallas import tpu_sc as plsc

_VOCAB = 100000
_MAXLEN = 200
_EMBED = 64
_BATCH = 1024
_NC = 2
_NS = 16
_NW = _NC * _NS
_CHUNK = 128
_NBB = _BATCH // _CHUNK
_NTC = _MAXLEN * _NBB
_NCHUNK = _NTC // _NW
_NBUF = 5
_LEAD = 3
_LANES = 16
_EV = _EMBED // 8
_TILE = 8 * _CHUNK
_COLW = _EMBED * _CHUNK
_LSTR = _EMBED * _BATCH
_OUTW = _MAXLEN * _LSTR
_PADW = _EMBED + 1
_XTL = _MAXLEN // 8


def _build():
    mesh = plsc.VectorSubcoreMesh(core_axis_name="c", subcore_axis_name="s")

    @functools.partial(
        pl.kernel,
        mesh=mesh,
        out_type=jax.ShapeDtypeStruct((_OUTW,), jnp.float32),
        scratch_types=[
            pltpu.VMEM((2, _NBB, 8, _CHUNK), jnp.int32),
            pltpu.VMEM((_NBUF, _CHUNK, _EMBED), jnp.float32),
            pltpu.VMEM((_CHUNK * _PADW,), jnp.float32),
            pltpu.VMEM((_NBUF, _COLW), jnp.float32),
            pltpu.VMEM((_EMBED, _MAXLEN + 8), jnp.float32),
            pltpu.SemaphoreType.DMA((_NBUF,)),
            pltpu.SemaphoreType.DMA((_NBUF,)),
        ],
        compiler_params=pltpu.CompilerParams(
            use_tc_tiling_on_sc=False, needs_layout_passes=False
        ),
    )
    def k(xq_hbm, tok_hbm, posd_hbm, out_hbm,
          idx_v, rows_v, pad_v, st_v, posv, gsem, ssem):
        wid = lax.axis_index("s") * _NC + lax.axis_index("c")
        g0 = wid * _NCHUNK
        l0 = g0 // _NBB
        tl0 = jnp.minimum(l0 // 8, _XTL - 2)
        pltpu.sync_copy(xq_hbm.at[pl.ds(tl0, 2)], idx_v)
        pltpu.sync_copy(posd_hbm, posv)

        iota = lax.iota(jnp.int32, _LANES)
        riota = [(iota + g * _LANES) * _PADW for g in range(_CHUNK // _LANES)]

        def start_gather(cc, b):
            g_tc = g0 + cc
            l = g_tc // _NBB
            tb = lax.rem(g_tc, _NBB)
            pltpu.async_copy(
                tok_hbm.at[idx_v.at[l // 8 - tl0, tb, lax.rem(l, 8)]],
                rows_v.at[b],
                gsem.at[b],
            )

        def wait_gather(b):
            pltpu.make_async_copy(
                out_hbm.at[pl.ds(0, _COLW)], rows_v.at[b], gsem.at[b]
            ).wait()

        def wait_store(b):
            pltpu.make_async_copy(
                st_v.at[b], out_hbm.at[pl.ds(0, _COLW)], ssem.at[b]
            ).wait()

        def compute(cc, b):
            l = (g0 + cc) // _NBB
            lcol = jnp.full((_LANES,), l, jnp.int32)
            pv = [plsc.load_gather(posv, [iota + t * _LANES, lcol])
                  for t in range(_EMBED // _LANES)]

            def j_body(j, c2):
                jb = j * _PADW
                vals = [
                    rows_v[b, j, pl.ds(t * _LANES, _LANES)] + pv[t]
                    for t in range(_EMBED // _LANES)
                ]
                for t in range(_EMBED // _LANES):
                    pad_v[pl.ds(jb + t * _LANES, _LANES)] = vals[t]
                return c2

            lax.fori_loop(0, _CHUNK, j_body, 0, unroll=8)

            def e_body(e, c2):
                ecol = jnp.full((_LANES,), e, jnp.int32)
                ebase = e * _CHUNK
                vs = [
                    plsc.load_gather(pad_v, [riota[g] + ecol])
                    for g in range(_CHUNK // _LANES)
                ]
                for g in range(_CHUNK // _LANES):
                    st_v[b, pl.ds(ebase + g * _LANES, _LANES)] = vs[g]
                return c2

            lax.fori_loop(0, _EMBED, e_body, 0, unroll=2)

        def start_store(cc, b):
            g_tc = g0 + cc
            l = g_tc // _NBB
            tb = lax.rem(g_tc, _NBB)
            cbase = l * _LSTR + tb * _TILE
            for te in range(_EV):
                pltpu.async_copy(
                    st_v.at[b, pl.ds(te * _TILE, _TILE)],
                    out_hbm.at[pl.ds(cbase + te * _NBB * _TILE, _TILE)],
                    ssem.at[b],
                )

        for b in range(_LEAD):
            start_gather(b, b)

        def body(cc, c):
            b = lax.rem(cc, _NBUF)
            nxt = cc + _LEAD
            bk = lax.rem(nxt, _NBUF)

            @pl.when(nxt < _NCHUNK)
            def _():
                @pl.when(nxt >= _NBUF)
                def _():
                    wait_store(bk)

                start_gather(nxt, bk)

            wait_gather(b)
            compute(cc, b)
            start_store(cc, b)
            return c

        lax.fori_loop(0, _NCHUNK, body, 0)
        for b in range(_NBUF):
            wait_store(b)

    return k


_k = _build()


def kernel(x, token_table, pos_table):
    xq = (
        x.astype(jnp.int32)
        .T.reshape(_XTL, 8, _NBB, _CHUNK)
        .transpose(0, 2, 1, 3)
    )
    posd = jnp.concatenate(
        [pos_table.T, jnp.zeros((_EMBED, 8), jnp.float32)], axis=1
    )
    out1d = _k(xq, token_table, posd)
    out = (
        out1d.reshape(_MAXLEN, _EV, _NBB, 8, _CHUNK)
        .transpose(2, 4, 0, 1, 3)
        .reshape(_BATCH, _MAXLEN, _EMBED)
    )
    return out


# speedup vs baseline: 1.0241x; 1.0241x over previous
import functools

import jax
import jax.numpy as jnp
from jax import lax
from jax.experimental import pallas as pl
from jax.experimental.pallas import tpu as pltpu
from jax.experimental.pallas import tpu_sc as plsc

_VOCAB = 100000
_MAXLEN = 200
_EMBED = 64
_BATCH = 1024
_NC = 2
_NS = 16
_NW = _NC * _NS
_CHUNK = 128
_NBB = _BATCH // _CHUNK
_NTC = _MAXLEN * _NBB
_NCHUNK = _NTC // _NW
_NBUF = 5
_LEAD = 3
_LANES = 16
_EV = _EMBED // 8
_TILE = 8 * _CHUNK
_COLW = _EMBED * _CHUNK
_LSTR = _EMBED * _BATCH
_OUTW = _MAXLEN * _LSTR
_PADW = _EMBED + 1
_XTL = _MAXLEN // 8


def _build():
    mesh = plsc.VectorSubcoreMesh(core_axis_name="c", subcore_axis_name="s")

    @functools.partial(
        pl.kernel,
        mesh=mesh,
        out_type=jax.ShapeDtypeStruct((_OUTW,), jnp.float32),
        scratch_types=[
            pltpu.VMEM((2, _NBB, 8, _CHUNK), jnp.int32),
            pltpu.VMEM((_NBUF, _CHUNK, _EMBED), jnp.float32),
            pltpu.VMEM((_CHUNK * _PADW,), jnp.float32),
            pltpu.VMEM((_NBUF, _COLW), jnp.float32),
            pltpu.VMEM((8, _EMBED), jnp.float32),
            pltpu.SemaphoreType.DMA((_NBUF,)),
            pltpu.SemaphoreType.DMA((_NBUF,)),
        ],
        compiler_params=pltpu.CompilerParams(
            use_tc_tiling_on_sc=False, needs_layout_passes=False
        ),
    )
    def k(xq_hbm, tok_hbm, posd_hbm, out_hbm,
          idx_v, rows_v, pad_v, st_v, posv, gsem, ssem):
        wid = lax.axis_index("s") * _NC + lax.axis_index("c")
        g0 = wid * _NCHUNK
        l0 = g0 // _NBB
        tl0 = jnp.minimum(l0 // 8, _XTL - 2)
        pltpu.sync_copy(xq_hbm.at[pl.ds(tl0, 2)], idx_v)
        pltpu.sync_copy(posd_hbm.at[pl.ds(l0, 8)], posv)

        iota = lax.iota(jnp.int32, _LANES)
        riota = [(iota + g * _LANES) * _PADW for g in range(_CHUNK // _LANES)]

        def start_gather(cc, b):
            g_tc = g0 + cc
            l = g_tc // _NBB
            tb = lax.rem(g_tc, _NBB)
            pltpu.async_copy(
                tok_hbm.at[idx_v.at[l // 8 - tl0, tb, lax.rem(l, 8)]],
                rows_v.at[b],
                gsem.at[b],
            )

        def wait_gather(b):
            pltpu.make_async_copy(
                out_hbm.at[pl.ds(0, _COLW)], rows_v.at[b], gsem.at[b]
            ).wait()

        def wait_store(b):
            pltpu.make_async_copy(
                st_v.at[b], out_hbm.at[pl.ds(0, _COLW)], ssem.at[b]
            ).wait()

        def compute(cc, b):
            l = (g0 + cc) // _NBB
            dl = l - l0
            pv = [posv[dl, pl.ds(t * _LANES, _LANES)]
                  for t in range(_EMBED // _LANES)]

            def j_body(j, c2):
                jb = j * _PADW
                vals = [
                    rows_v[b, j, pl.ds(t * _LANES, _LANES)] + pv[t]
                    for t in range(_EMBED // _LANES)
                ]
                for t in range(_EMBED // _LANES):
                    pad_v[pl.ds(jb + t * _LANES, _LANES)] = vals[t]
                return c2

            lax.fori_loop(0, _CHUNK, j_body, 0, unroll=8)

            def e_body(e, c2):
                ecol = jnp.full((_LANES,), e, jnp.int32)
                ebase = e * _CHUNK
                vs = [
                    plsc.load_gather(pad_v, [riota[g] + ecol])
                    for g in range(_CHUNK // _LANES)
                ]
                for g in range(_CHUNK // _LANES):
                    st_v[b, pl.ds(ebase + g * _LANES, _LANES)] = vs[g]
                return c2

            lax.fori_loop(0, _EMBED, e_body, 0, unroll=2)

        def start_store(cc, b):
            g_tc = g0 + cc
            l = g_tc // _NBB
            tb = lax.rem(g_tc, _NBB)
            cbase = l * _LSTR + tb * _TILE
            for te in range(_EV):
                pltpu.async_copy(
                    st_v.at[b, pl.ds(te * _TILE, _TILE)],
                    out_hbm.at[pl.ds(cbase + te * _NBB * _TILE, _TILE)],
                    ssem.at[b],
                )

        for b in range(_LEAD):
            start_gather(b, b)

        def body(cc, c):
            b = lax.rem(cc, _NBUF)
            nxt = cc + _LEAD
            bk = lax.rem(nxt, _NBUF)

            @pl.when(nxt < _NCHUNK)
            def _():
                @pl.when(nxt >= _NBUF)
                def _():
                    wait_store(bk)

                start_gather(nxt, bk)

            wait_gather(b)
            compute(cc, b)
            start_store(cc, b)
            return c

        lax.fori_loop(0, _NCHUNK, body, 0)
        for b in range(_NBUF):
            wait_store(b)

    return k


_k = _build()


def kernel(x, token_table, pos_table):
    xq = (
        x.astype(jnp.int32)
        .T.reshape(_XTL, 8, _NBB, _CHUNK)
        .transpose(0, 2, 1, 3)
    )
    posd = jnp.concatenate(
        [pos_table, jnp.zeros((8, _EMBED), jnp.float32)], axis=0
    )
    out1d = _k(xq, token_table, posd)
    out = (
        out1d.reshape(_MAXLEN, _EV, _NBB, 8, _CHUNK)
        .transpose(2, 4, 0, 1, 3)
        .reshape(_BATCH, _MAXLEN, _EMBED)
    )
    return out
